# baseline (speedup 1.0000x reference)
import jax
import jax.numpy as jnp
from jax import lax
from jax.experimental import pallas as pl
from jax.experimental.pallas import tpu as pltpu
from jax.experimental.pallas import tpu_sc as plsc

R = 32
A = 8
B = 32768
NW = 32
BPW = B // NW
GROUPS = BPW // 16
L = 16


def _body(x_hbm, sig_hbm, cc_hbm, cw_hbm, out_hbm,
          xv, sig_v, cc_v, cw_v, num_v, den_v):
    wid = lax.axis_index("s") * 2 + lax.axis_index("c")
    base = wid * BPW

    for k in range(A):
        pltpu.sync_copy(x_hbm.at[k, pl.ds(base, BPW)], xv.at[k])
    pltpu.sync_copy(sig_hbm, sig_v)
    pltpu.sync_copy(cc_hbm, cc_v)
    pltpu.sync_copy(cw_hbm, cw_v)

    for i in range(R):
        na = []
        cc = []
        for k in range(A):
            s = sig_v[A * i + k, :]
            na.append(-0.5 / (s * s))
            cc.append(cc_v[A * i + k, :])
        cw = [cw_v[(A + 1) * i + j, :] for j in range(A + 1)]

        def grp(g, carry, na=na, cc=cc, cw=cw, rule=i):
            sl = pl.ds(g * L, L)
            acc = None
            ch = cw[0]
            for k in range(A):
                xk = xv[k, sl]
                d = xk - cc[k]
                t = na[k] * (d * d)
                acc = t if acc is None else acc + t
                ch = ch + cw[k + 1] * xk
            uu = jnp.exp(acc)
            if rule == 0:
                num_v[sl] = uu * ch
                den_v[sl] = uu
            else:
                plsc.addupdate(num_v.at[sl], uu * ch)
                plsc.addupdate(den_v.at[sl], uu)
            return carry

        lax.fori_loop(0, GROUPS, grp, 0)

    def fin(g, carry):
        sl = pl.ds(g * L, L)
        num_v[sl] = num_v[sl] / den_v[sl]
        return carry

    lax.fori_loop(0, GROUPS, fin, 0)

    pltpu.sync_copy(num_v, out_hbm.at[pl.ds(base, BPW)])


@jax.jit
def kernel(input_data, FRB_weights, C):
    xT = input_data.T
    idx = jnp.arange(R * A)
    sig = jnp.tile(jnp.take(FRB_weights, idx)[:, None], (1, L))
    cc = jnp.tile(jnp.take(FRB_weights, idx + 1)[:, None], (1, L))
    cw = jnp.tile(C.reshape(-1)[:, None], (1, L))

    mesh = plsc.VectorSubcoreMesh(core_axis_name="c", subcore_axis_name="s")
    run = pl.kernel(
        _body,
        out_type=jax.ShapeDtypeStruct((B,), jnp.float32),
        mesh=mesh,
        scratch_types=[
            pltpu.VMEM((A, BPW), jnp.float32),
            pltpu.VMEM((R * A, L), jnp.float32),
            pltpu.VMEM((R * A, L), jnp.float32),
            pltpu.VMEM((R * (A + 1), L), jnp.float32),
            pltpu.VMEM((BPW,), jnp.float32),
            pltpu.VMEM((BPW,), jnp.float32),
        ],
    )
    return run(xT, sig, cc, cw)


# speedup vs baseline: 1.5064x; 1.5064x over previous
import jax
import jax.numpy as jnp
from jax import lax
from jax.experimental import pallas as pl
from jax.experimental.pallas import tpu as pltpu
from jax.experimental.pallas import tpu_sc as plsc

R = 32
A = 8
B = 32768
NW = 32
BPW = B // NW
GROUPS = BPW // 16
L = 16


def _tree8(t):
    return ((t[0] + t[1]) + (t[2] + t[3])) + ((t[4] + t[5]) + (t[6] + t[7]))


def _body(x_hbm, sig_hbm, cc_hbm, cw_hbm, out_hbm,
          xv, sig_v, cc_v, cw_v, num_v, den_v, sem):
    wid = lax.axis_index("s") * 2 + lax.axis_index("c")
    base = wid * BPW

    copies = [pltpu.async_copy(x_hbm.at[k, pl.ds(base, BPW)], xv.at[k], sem)
              for k in range(A)]
    copies.append(pltpu.async_copy(sig_hbm, sig_v, sem))
    copies.append(pltpu.async_copy(cc_hbm, cc_v, sem))
    copies.append(pltpu.async_copy(cw_hbm, cw_v, sem))
    for c in copies:
        c.wait()

    for i in range(R):
        na = []
        cc = []
        for k in range(A):
            s = sig_v[A * i + k, :]
            na.append(-0.5 / (s * s))
            cc.append(cc_v[A * i + k, :])
        cw = [cw_v[(A + 1) * i + j, :] for j in range(A + 1)]

        def grp(g, na=na, cc=cc, cw=cw, rule=i):
            sl = pl.ds(g * L, L)
            xs = [xv[k, sl] for k in range(A)]
            d = [xs[k] - cc[k] for k in range(A)]
            acc = _tree8([na[k] * (d[k] * d[k]) for k in range(A)])
            ch = cw[0] + _tree8([cw[k + 1] * xs[k] for k in range(A)])
            uu = jnp.exp(acc)
            if rule == 0:
                num_v[sl] = uu * ch
                den_v[sl] = uu
            else:
                plsc.addupdate(num_v.at[sl], uu * ch)
                plsc.addupdate(den_v.at[sl], uu)

        plsc.parallel_loop(0, GROUPS, unroll=1)(grp)

    @plsc.parallel_loop(0, GROUPS, unroll=1)
    def fin(g):
        sl = pl.ds(g * L, L)
        num_v[sl] = num_v[sl] / den_v[sl]

    pltpu.sync_copy(num_v, out_hbm.at[pl.ds(base, BPW)])


@jax.jit
def kernel(input_data, FRB_weights, C):
    xT = input_data.T
    idx = jnp.arange(R * A)
    sig = jnp.tile(jnp.take(FRB_weights, idx)[:, None], (1, L))
    cc = jnp.tile(jnp.take(FRB_weights, idx + 1)[:, None], (1, L))
    cw = jnp.tile(C.reshape(-1)[:, None], (1, L))

    mesh = plsc.VectorSubcoreMesh(core_axis_name="c", subcore_axis_name="s")
    run = pl.kernel(
        _body,
        out_type=jax.ShapeDtypeStruct((B,), jnp.float32),
        mesh=mesh,
        scratch_types=[
            pltpu.VMEM((A, BPW), jnp.float32),
            pltpu.VMEM((R * A, L), jnp.float32),
            pltpu.VMEM((R * A, L), jnp.float32),
            pltpu.VMEM((R * (A + 1), L), jnp.float32),
            pltpu.VMEM((BPW,), jnp.float32),
            pltpu.VMEM((BPW,), jnp.float32),
            pltpu.SemaphoreType.DMA,
        ],
    )
    return run(xT, sig, cc, cw)


# speedup vs baseline: 2.0259x; 1.3449x over previous
import jax
import jax.numpy as jnp
from jax import lax
from jax.experimental import pallas as pl
from jax.experimental.pallas import tpu as pltpu
from jax.experimental.pallas import tpu_sc as plsc

R = 32
A = 8
B = 32768
NW = 32
B_SC = 4096
B_TC = B - B_SC
BT = 4096
BPW = B_SC // NW
GROUPS = BPW // 16
L = 16


def _tree8(t):
    return ((t[0] + t[1]) + (t[2] + t[3])) + ((t[4] + t[5]) + (t[6] + t[7]))


def _body(x_hbm, sig_hbm, cc_hbm, cw_hbm, out_hbm,
          xv, sig_v, cc_v, cw_v, num_v, den_v, sem):
    wid = lax.axis_index("s") * 2 + lax.axis_index("c")
    base = wid * BPW

    copies = [pltpu.async_copy(x_hbm.at[k, pl.ds(base, BPW)], xv.at[k], sem)
              for k in range(A)]
    copies.append(pltpu.async_copy(sig_hbm, sig_v, sem))
    copies.append(pltpu.async_copy(cc_hbm, cc_v, sem))
    copies.append(pltpu.async_copy(cw_hbm, cw_v, sem))
    for c in copies:
        c.wait()

    for i in range(R):
        na = []
        cc = []
        for k in range(A):
            s = sig_v[A * i + k, :]
            na.append(-0.5 / (s * s))
            cc.append(cc_v[A * i + k, :])
        cw = [cw_v[(A + 1) * i + j, :] for j in range(A + 1)]

        def grp(g, na=na, cc=cc, cw=cw, rule=i):
            sl = pl.ds(g * L, L)
            xs = [xv[k, sl] for k in range(A)]
            d = [xs[k] - cc[k] for k in range(A)]
            acc = _tree8([na[k] * (d[k] * d[k]) for k in range(A)])
            ch = cw[0] + _tree8([cw[k + 1] * xs[k] for k in range(A)])
            uu = jnp.exp(acc)
            if rule == 0:
                num_v[sl] = uu * ch
                den_v[sl] = uu
            else:
                plsc.addupdate(num_v.at[sl], uu * ch)
                plsc.addupdate(den_v.at[sl], uu)

        plsc.parallel_loop(0, GROUPS, unroll=1)(grp)

    @plsc.parallel_loop(0, GROUPS, unroll=1)
    def fin(g):
        sl = pl.ds(g * L, L)
        num_v[sl] = num_v[sl] / den_v[sl]

    pltpu.sync_copy(num_v, out_hbm.at[pl.ds(base, BPW)])


def _tc_body(xt_ref, sig_ref, cc_ref, c_ref, out_ref):
    x = xt_ref[:]
    sig = sig_ref[:]
    cc = cc_ref[:]
    Cm = c_ref[:]
    na = -0.5 / (sig * sig)
    w1 = -2.0 * na * cc
    w0 = jnp.sum(na * cc * cc, axis=1, keepdims=True)
    logUU = (jnp.dot(na, x * x, preferred_element_type=jnp.float32)
             + jnp.dot(w1, x, preferred_element_type=jnp.float32)
             + w0)
    UU = jnp.exp(logUU)
    CH = (jnp.dot(Cm[:, 1:], x, preferred_element_type=jnp.float32)
          + Cm[:, 0][:, None])
    num = jnp.sum(UU * CH, axis=0, keepdims=True)
    den = jnp.sum(UU, axis=0, keepdims=True)
    out_ref[:] = num / den


@jax.jit
def kernel(input_data, FRB_weights, C):
    xT = input_data.T
    idx = jnp.arange(R * A)
    sig2 = jnp.take(FRB_weights, idx).reshape(R, A)
    cc2 = jnp.take(FRB_weights, idx + 1).reshape(R, A)
    sig = jnp.tile(sig2.reshape(-1)[:, None], (1, L))
    cc = jnp.tile(cc2.reshape(-1)[:, None], (1, L))
    cw = jnp.tile(C.reshape(-1)[:, None], (1, L))

    mesh = plsc.VectorSubcoreMesh(core_axis_name="c", subcore_axis_name="s")
    run_sc = pl.kernel(
        _body,
        out_type=jax.ShapeDtypeStruct((B_SC,), jnp.float32),
        mesh=mesh,
        scratch_types=[
            pltpu.VMEM((A, BPW), jnp.float32),
            pltpu.VMEM((R * A, L), jnp.float32),
            pltpu.VMEM((R * A, L), jnp.float32),
            pltpu.VMEM((R * (A + 1), L), jnp.float32),
            pltpu.VMEM((BPW,), jnp.float32),
            pltpu.VMEM((BPW,), jnp.float32),
            pltpu.SemaphoreType.DMA,
        ],
    )
    sc_out = run_sc(xT[:, :B_SC], sig, cc, cw)

    run_tc = pl.pallas_call(
        _tc_body,
        grid=(B_TC // BT,),
        in_specs=[
            pl.BlockSpec((A, BT), lambda i: (0, i)),
            pl.BlockSpec((R, A), lambda i: (0, 0)),
            pl.BlockSpec((R, A), lambda i: (0, 0)),
            pl.BlockSpec((R, A + 1), lambda i: (0, 0)),
        ],
        out_specs=pl.BlockSpec((1, BT), lambda i: (0, i)),
        out_shape=jax.ShapeDtypeStruct((1, B_TC), jnp.float32),
    )
    tc_out = run_tc(xT[:, B_SC:], sig2, cc2, C)[0]

    return jnp.concatenate([sc_out, tc_out])


# speedup vs baseline: 2.2500x; 1.1106x over previous
import jax
import jax.numpy as jnp
from jax import lax
from jax.experimental import pallas as pl
from jax.experimental.pallas import tpu as pltpu
from jax.experimental.pallas import tpu_sc as plsc

R = 32
A = 8
B = 32768
NC = 1
NW = 16 * NC
B_SC = 4096
B_TC = B - B_SC
BT = 4096
BPW = B_SC // NW
GROUPS = BPW // 16
L = 16


def _tree8(t):
    return ((t[0] + t[1]) + (t[2] + t[3])) + ((t[4] + t[5]) + (t[6] + t[7]))


def _body(x_hbm, sig_hbm, cc_hbm, cw_hbm, out_hbm,
          xv, sig_v, cc_v, cw_v, num_v, den_v, sem):
    wid = lax.axis_index("s") * NC + lax.axis_index("c")
    base = wid * BPW

    copies = [pltpu.async_copy(x_hbm.at[k, pl.ds(base, BPW)], xv.at[k], sem)
              for k in range(A)]
    copies.append(pltpu.async_copy(sig_hbm, sig_v, sem))
    copies.append(pltpu.async_copy(cc_hbm, cc_v, sem))
    copies.append(pltpu.async_copy(cw_hbm, cw_v, sem))
    for c in copies:
        c.wait()

    for i in range(R):
        na = []
        cc = []
        for k in range(A):
            s = sig_v[A * i + k, :]
            na.append(-0.5 / (s * s))
            cc.append(cc_v[A * i + k, :])
        cw = [cw_v[(A + 1) * i + j, :] for j in range(A + 1)]

        def grp(g, na=na, cc=cc, cw=cw, rule=i):
            sl = pl.ds(g * L, L)
            xs = [xv[k, sl] for k in range(A)]
            d = [xs[k] - cc[k] for k in range(A)]
            acc = _tree8([na[k] * (d[k] * d[k]) for k in range(A)])
            ch = cw[0] + _tree8([cw[k + 1] * xs[k] for k in range(A)])
            uu = jnp.exp(acc)
            if rule == 0:
                num_v[sl] = uu * ch
                den_v[sl] = uu
            else:
                plsc.addupdate(num_v.at[sl], uu * ch)
                plsc.addupdate(den_v.at[sl], uu)

        plsc.parallel_loop(0, GROUPS, unroll=1)(grp)

    @plsc.parallel_loop(0, GROUPS, unroll=1)
    def fin(g):
        sl = pl.ds(g * L, L)
        num_v[sl] = num_v[sl] / den_v[sl]

    pltpu.sync_copy(num_v, out_hbm.at[pl.ds(base, BPW)])


def _tc_body(xt_ref, sig_ref, cc_ref, c_ref, out_ref):
    x = xt_ref[:]
    sig = sig_ref[:]
    cc = cc_ref[:]
    Cm = c_ref[:]
    na = -0.5 / (sig * sig)
    w1 = -2.0 * na * cc
    w0 = jnp.sum(na * cc * cc, axis=1, keepdims=True)
    logUU = (jnp.dot(na, x * x, preferred_element_type=jnp.float32)
             + jnp.dot(w1, x, preferred_element_type=jnp.float32)
             + w0)
    UU = jnp.exp(logUU)
    CH = (jnp.dot(Cm[:, 1:], x, preferred_element_type=jnp.float32)
          + Cm[:, 0][:, None])
    num = jnp.sum(UU * CH, axis=0, keepdims=True)
    den = jnp.sum(UU, axis=0, keepdims=True)
    out_ref[:] = num / den


@jax.jit
def kernel(input_data, FRB_weights, C):
    xT = input_data.T
    idx = jnp.arange(R * A)
    sig2 = jnp.take(FRB_weights, idx).reshape(R, A)
    cc2 = jnp.take(FRB_weights, idx + 1).reshape(R, A)
    sig = jnp.tile(sig2.reshape(-1)[:, None], (1, L))
    cc = jnp.tile(cc2.reshape(-1)[:, None], (1, L))
    cw = jnp.tile(C.reshape(-1)[:, None], (1, L))

    mesh = plsc.VectorSubcoreMesh(core_axis_name="c", subcore_axis_name="s",
                                  num_cores=NC)
    run_sc = pl.kernel(
        _body,
        out_type=jax.ShapeDtypeStruct((B_SC,), jnp.float32),
        mesh=mesh,
        scratch_types=[
            pltpu.VMEM((A, BPW), jnp.float32),
            pltpu.VMEM((R * A, L), jnp.float32),
            pltpu.VMEM((R * A, L), jnp.float32),
            pltpu.VMEM((R * (A + 1), L), jnp.float32),
            pltpu.VMEM((BPW,), jnp.float32),
            pltpu.VMEM((BPW,), jnp.float32),
            pltpu.SemaphoreType.DMA,
        ],
    )
    sc_out = run_sc(xT[:, :B_SC], sig, cc, cw)

    run_tc = pl.pallas_call(
        _tc_body,
        grid=(B_TC // BT,),
        in_specs=[
            pl.BlockSpec((A, BT), lambda i: (0, i)),
            pl.BlockSpec((R, A), lambda i: (0, 0)),
            pl.BlockSpec((R, A), lambda i: (0, 0)),
            pl.BlockSpec((R, A + 1), lambda i: (0, 0)),
        ],
        out_specs=pl.BlockSpec((1, BT), lambda i: (0, i)),
        out_shape=jax.ShapeDtypeStruct((1, B_TC), jnp.float32),
    )
    tc_out = run_tc(xT[:, B_SC:], sig2, cc2, C)[0]

    return jnp.concatenate([sc_out, tc_out])


# speedup vs baseline: 2.3057x; 1.0248x over previous
import jax
import jax.numpy as jnp
from jax import lax
from jax.experimental import pallas as pl
from jax.experimental.pallas import tpu as pltpu
from jax.experimental.pallas import tpu_sc as plsc

R = 32
A = 8
B = 32768
NC = 1
NW = 16 * NC
B_SC = 4096
B_TC = B - B_SC
BT = 4096
BPW = B_SC // NW
GROUPS = BPW // 16
L = 16


def _tree8(t):
    return ((t[0] + t[1]) + (t[2] + t[3])) + ((t[4] + t[5]) + (t[6] + t[7]))


def _body(x_hbm, sig_hbm, cc_hbm, cw_hbm, out_hbm,
          xv, sig_v, cc_v, cw_v, num_v, den_v, sem):
    wid = lax.axis_index("s") * NC + lax.axis_index("c")
    base = wid * BPW

    copies = [pltpu.async_copy(x_hbm.at[k, pl.ds(base, BPW)], xv.at[k], sem)
              for k in range(A)]
    copies.append(pltpu.async_copy(sig_hbm, sig_v, sem))
    copies.append(pltpu.async_copy(cc_hbm, cc_v, sem))
    copies.append(pltpu.async_copy(cw_hbm, cw_v, sem))
    for c in copies:
        c.wait()

    for i in range(R):
        na = []
        cc = []
        for k in range(A):
            s = sig_v[A * i + k, :]
            na.append(-0.5 / (s * s))
            cc.append(cc_v[A * i + k, :])
        cw = [cw_v[(A + 1) * i + j, :] for j in range(A + 1)]

        def grp(g, na=na, cc=cc, cw=cw, rule=i):
            sl = pl.ds(g * L, L)
            xs = [xv[k, sl] for k in range(A)]
            d = [xs[k] - cc[k] for k in range(A)]
            acc = _tree8([na[k] * (d[k] * d[k]) for k in range(A)])
            ch = cw[0] + _tree8([cw[k + 1] * xs[k] for k in range(A)])
            uu = jnp.exp(acc)
            if rule == 0:
                num_v[sl] = uu * ch
                den_v[sl] = uu
            else:
                plsc.addupdate(num_v.at[sl], uu * ch)
                plsc.addupdate(den_v.at[sl], uu)

        plsc.parallel_loop(0, GROUPS, unroll=1)(grp)

    @plsc.parallel_loop(0, GROUPS, unroll=1)
    def fin(g):
        sl = pl.ds(g * L, L)
        num_v[sl] = num_v[sl] / den_v[sl]

    pltpu.sync_copy(num_v, out_hbm.at[pl.ds(base, BPW)])


def _tc_body(xt_ref, sig_ref, cc_ref, c_ref, out_ref):
    x = xt_ref[:]
    sig = sig_ref[:]
    cc = cc_ref[:]
    Cm = c_ref[:]
    na = -0.5 / (sig * sig)
    w1 = -2.0 * na * cc
    w0 = jnp.sum(na * cc * cc, axis=1, keepdims=True)
    logUU = (jnp.dot(na, x * x, preferred_element_type=jnp.float32)
             + jnp.dot(w1, x, preferred_element_type=jnp.float32)
             + w0)
    UU = jnp.exp(logUU)
    CH = (jnp.dot(Cm[:, 1:], x, preferred_element_type=jnp.float32)
          + Cm[:, 0][:, None])
    num = jnp.sum(UU * CH, axis=0, keepdims=True)
    den = jnp.sum(UU, axis=0, keepdims=True)
    out_ref[:] = num / den


@jax.jit
def kernel(input_data, FRB_weights, C):
    xT = input_data.T
    sig1 = lax.slice(FRB_weights, (0,), (R * A,))
    cc1 = lax.slice(FRB_weights, (1,), (R * A + 1,))
    sig2 = sig1.reshape(R, A)
    cc2 = cc1.reshape(R, A)
    sig = jnp.broadcast_to(sig1[:, None], (R * A, L))
    cc = jnp.broadcast_to(cc1[:, None], (R * A, L))
    cw = jnp.broadcast_to(C.reshape(-1)[:, None], (R * (A + 1), L))

    mesh = plsc.VectorSubcoreMesh(core_axis_name="c", subcore_axis_name="s",
                                  num_cores=NC)
    run_sc = pl.kernel(
        _body,
        out_type=jax.ShapeDtypeStruct((B_SC,), jnp.float32),
        mesh=mesh,
        scratch_types=[
            pltpu.VMEM((A, BPW), jnp.float32),
            pltpu.VMEM((R * A, L), jnp.float32),
            pltpu.VMEM((R * A, L), jnp.float32),
            pltpu.VMEM((R * (A + 1), L), jnp.float32),
            pltpu.VMEM((BPW,), jnp.float32),
            pltpu.VMEM((BPW,), jnp.float32),
            pltpu.SemaphoreType.DMA,
        ],
    )
    sc_out = run_sc(xT[:, :B_SC], sig, cc, cw)

    run_tc = pl.pallas_call(
        _tc_body,
        grid=(B_TC // BT,),
        in_specs=[
            pl.BlockSpec((A, BT), lambda i: (0, i)),
            pl.BlockSpec((R, A), lambda i: (0, 0)),
            pl.BlockSpec((R, A), lambda i: (0, 0)),
            pl.BlockSpec((R, A + 1), lambda i: (0, 0)),
        ],
        out_specs=pl.BlockSpec((1, BT), lambda i: (0, i)),
        out_shape=jax.ShapeDtypeStruct((1, B_TC), jnp.float32),
    )
    tc_out = run_tc(xT[:, B_SC:], sig2, cc2, C)[0]

    return jnp.concatenate([sc_out, tc_out])


# speedup vs baseline: 2.5776x; 1.1179x over previous
import jax
import jax.numpy as jnp
from jax import lax
from jax.experimental import pallas as pl
from jax.experimental.pallas import tpu as pltpu
from jax.experimental.pallas import tpu_sc as plsc

R = 32
A = 8
B = 32768
NC = 1
NW = 16 * NC
B_SC = 2048
B_TC = B - B_SC
BT = 10240
BPW = B_SC // NW
GROUPS = BPW // 16
L = 16
TAB = 2 * R * A + R * (A + 1)


def _tree8(t):
    return ((t[0] + t[1]) + (t[2] + t[3])) + ((t[4] + t[5]) + (t[6] + t[7]))


def _sc_body(x_hbm, tab_hbm, out_hbm, xv, tab_v, num_v, den_v, sem):
    wid = lax.axis_index("s") * NC + lax.axis_index("c")
    base = B_TC + wid * BPW

    copies = [pltpu.async_copy(x_hbm.at[k, pl.ds(base, BPW)], xv.at[k], sem)
              for k in range(A)]
    copies.append(pltpu.async_copy(tab_hbm, tab_v, sem))
    for c in copies:
        c.wait()

    for i in range(R):
        na = []
        cc = []
        for k in range(A):
            s = tab_v[A * i + k, :]
            na.append(-0.5 / (s * s))
            cc.append(tab_v[R * A + A * i + k, :])
        cw = [tab_v[2 * R * A + (A + 1) * i + j, :] for j in range(A + 1)]

        def grp(g, na=na, cc=cc, cw=cw, rule=i):
            sl = pl.ds(g * L, L)
            xs = [xv[k, sl] for k in range(A)]
            d = [xs[k] - cc[k] for k in range(A)]
            acc = _tree8([na[k] * (d[k] * d[k]) for k in range(A)])
            ch = cw[0] + _tree8([cw[k + 1] * xs[k] for k in range(A)])
            uu = jnp.exp(acc)
            if rule == 0:
                num_v[sl] = uu * ch
                den_v[sl] = uu
            else:
                plsc.addupdate(num_v.at[sl], uu * ch)
                plsc.addupdate(den_v.at[sl], uu)

        plsc.parallel_loop(0, GROUPS, unroll=1)(grp)

    @plsc.parallel_loop(0, GROUPS, unroll=1)
    def fin(g):
        sl = pl.ds(g * L, L)
        num_v[sl] = num_v[sl] / den_v[sl]

    pltpu.sync_copy(num_v, out_hbm.at[pl.ds(wid * BPW, BPW)])


def _tc_body(xt_ref, sig_ref, cc_ref, c_ref, out_ref):
    x = xt_ref[:]
    sig = sig_ref[:]
    cc = cc_ref[:]
    Cm = c_ref[:]
    na = -0.5 / (sig * sig)
    w1 = -2.0 * na * cc
    w0 = jnp.sum(na * cc * cc, axis=1, keepdims=True)
    logUU = (jnp.dot(na, x * x, preferred_element_type=jnp.float32)
             + jnp.dot(w1, x, preferred_element_type=jnp.float32)
             + w0)
    UU = jnp.exp(logUU)
    CH = (jnp.dot(Cm[:, 1:], x, preferred_element_type=jnp.float32)
          + Cm[:, 0][:, None])
    ones = jnp.ones((1, R), jnp.float32)
    num = jnp.dot(ones, UU * CH, preferred_element_type=jnp.float32)
    den = jnp.dot(ones, UU, preferred_element_type=jnp.float32)
    out_ref[:] = (num / den)[0]


@jax.jit
def kernel(input_data, FRB_weights, C):
    xT = input_data.T
    sig1 = lax.slice(FRB_weights, (0,), (R * A,))
    cc1 = lax.slice(FRB_weights, (1,), (R * A + 1,))
    tab = jnp.broadcast_to(
        jnp.concatenate([sig1, cc1, C.reshape(-1)])[:, None], (TAB, L))

    mesh = plsc.VectorSubcoreMesh(core_axis_name="c", subcore_axis_name="s",
                                  num_cores=NC)
    run_sc = pl.kernel(
        _sc_body,
        out_type=jax.ShapeDtypeStruct((B_SC,), jnp.float32),
        mesh=mesh,
        scratch_types=[
            pltpu.VMEM((A, BPW), jnp.float32),
            pltpu.VMEM((TAB, L), jnp.float32),
            pltpu.VMEM((BPW,), jnp.float32),
            pltpu.VMEM((BPW,), jnp.float32),
            pltpu.SemaphoreType.DMA,
        ],
    )
    sc_out = run_sc(xT, tab)

    sig2 = sig1.reshape(R, A)
    cc2 = cc1.reshape(R, A)
    run_tc = pl.pallas_call(
        _tc_body,
        grid=(B_TC // BT,),
        in_specs=[
            pl.BlockSpec((A, BT), lambda i: (0, i)),
            pl.BlockSpec((R, A), lambda i: (0, 0)),
            pl.BlockSpec((R, A), lambda i: (0, 0)),
            pl.BlockSpec((R, A + 1), lambda i: (0, 0)),
        ],
        out_specs=pl.BlockSpec((BT,), lambda i: (i,)),
        out_shape=jax.ShapeDtypeStruct((B_TC,), jnp.float32),
    )
    tc_out = run_tc(xT, sig2, cc2, C)

    return jnp.concatenate([tc_out, sc_out])


# speedup vs baseline: 2.6336x; 1.0217x over previous
import jax
import jax.numpy as jnp
from jax import lax
from jax.experimental import pallas as pl
from jax.experimental.pallas import tpu as pltpu
from jax.experimental.pallas import tpu_sc as plsc

R = 32
A = 8
B = 32768
NC = 1
NW = 16 * NC
B_SC = 1024
B_TC = B - B_SC
BT = 31744
BPW = B_SC // NW
GROUPS = BPW // 16
L = 16
TAB = 2 * R * A + R * (A + 1)


def _tree8(t):
    return ((t[0] + t[1]) + (t[2] + t[3])) + ((t[4] + t[5]) + (t[6] + t[7]))


def _sc_body(x_hbm, tab_hbm, out_hbm, xv, tab_v, num_v, den_v, sem):
    wid = lax.axis_index("s") * NC + lax.axis_index("c")
    base = B_TC + wid * BPW

    copies = [pltpu.async_copy(x_hbm.at[k, pl.ds(base, BPW)], xv.at[k], sem)
              for k in range(A)]
    copies.append(pltpu.async_copy(tab_hbm, tab_v, sem))
    for c in copies:
        c.wait()

    for i in range(R):
        na = []
        cc = []
        for k in range(A):
            s = tab_v[A * i + k, :]
            na.append(-0.5 / (s * s))
            cc.append(tab_v[R * A + A * i + k, :])
        cw = [tab_v[2 * R * A + (A + 1) * i + j, :] for j in range(A + 1)]

        def grp(g, na=na, cc=cc, cw=cw, rule=i):
            sl = pl.ds(g * L, L)
            xs = [xv[k, sl] for k in range(A)]
            d = [xs[k] - cc[k] for k in range(A)]
            acc = _tree8([na[k] * (d[k] * d[k]) for k in range(A)])
            ch = cw[0] + _tree8([cw[k + 1] * xs[k] for k in range(A)])
            uu = jnp.exp(acc)
            if rule == 0:
                num_v[sl] = uu * ch
                den_v[sl] = uu
            else:
                plsc.addupdate(num_v.at[sl], uu * ch)
                plsc.addupdate(den_v.at[sl], uu)

        plsc.parallel_loop(0, GROUPS, unroll=1)(grp)

    @plsc.parallel_loop(0, GROUPS, unroll=1)
    def fin(g):
        sl = pl.ds(g * L, L)
        num_v[sl] = num_v[sl] / den_v[sl]

    pltpu.sync_copy(num_v, out_hbm.at[pl.ds(wid * BPW, BPW)])


def _tc_body(xt_ref, sig_ref, cc_ref, c_ref, out_ref):
    x = xt_ref[:]
    sig = sig_ref[:]
    cc = cc_ref[:]
    Cm = c_ref[:]
    na = -0.5 / (sig * sig)
    w1 = -2.0 * na * cc
    w0 = jnp.sum(na * cc * cc, axis=1, keepdims=True)
    logUU = (jnp.dot(na, x * x, preferred_element_type=jnp.float32)
             + jnp.dot(w1, x, preferred_element_type=jnp.float32)
             + w0)
    UU = jnp.exp(logUU)
    CH = (jnp.dot(Cm[:, 1:], x, preferred_element_type=jnp.float32)
          + Cm[:, 0][:, None])
    ones = jnp.ones((1, R), jnp.float32)
    num = jnp.dot(ones, UU * CH, preferred_element_type=jnp.float32)
    den = jnp.dot(ones, UU, preferred_element_type=jnp.float32)
    out_ref[:] = (num / den)[0]


@jax.jit
def kernel(input_data, FRB_weights, C):
    xT = input_data.T
    sig1 = lax.slice(FRB_weights, (0,), (R * A,))
    cc1 = lax.slice(FRB_weights, (1,), (R * A + 1,))
    tab = jnp.broadcast_to(
        jnp.concatenate([sig1, cc1, C.reshape(-1)])[:, None], (TAB, L))

    mesh = plsc.VectorSubcoreMesh(core_axis_name="c", subcore_axis_name="s",
                                  num_cores=NC)
    run_sc = pl.kernel(
        _sc_body,
        out_type=jax.ShapeDtypeStruct((B_SC,), jnp.float32),
        mesh=mesh,
        scratch_types=[
            pltpu.VMEM((A, BPW), jnp.float32),
            pltpu.VMEM((TAB, L), jnp.float32),
            pltpu.VMEM((BPW,), jnp.float32),
            pltpu.VMEM((BPW,), jnp.float32),
            pltpu.SemaphoreType.DMA,
        ],
    )
    sc_out = run_sc(xT, tab)

    sig2 = sig1.reshape(R, A)
    cc2 = cc1.reshape(R, A)
    run_tc = pl.pallas_call(
        _tc_body,
        grid=(B_TC // BT,),
        in_specs=[
            pl.BlockSpec((A, BT), lambda i: (0, i)),
            pl.BlockSpec((R, A), lambda i: (0, 0)),
            pl.BlockSpec((R, A), lambda i: (0, 0)),
            pl.BlockSpec((R, A + 1), lambda i: (0, 0)),
        ],
        out_specs=pl.BlockSpec((BT,), lambda i: (i,)),
        out_shape=jax.ShapeDtypeStruct((B_TC,), jnp.float32),
    )
    tc_out = run_tc(xT, sig2, cc2, C)

    return jnp.concatenate([tc_out, sc_out])
